# baseline (speedup 1.0000x reference)
import functools
import math

import jax
import jax.numpy as jnp
from jax import lax
from jax.experimental import pallas as pl
from jax.experimental.pallas import tpu as pltpu
from jax.experimental.pallas import tpu_sc as plsc

_NC = 2
_NS = 16
_NW = _NC * _NS
_LANES = 16
_CHUNK = 320
_DPAD = 128


def _make_embed(n_rows: int, d: int):
    assert n_rows % _NW == 0
    rows_per_w = n_rows // _NW
    assert rows_per_w % (2 * _CHUNK) == 0
    n_pairs = rows_per_w // (2 * _CHUNK)
    scale = jnp.float32(math.sqrt(d))
    mesh = plsc.VectorSubcoreMesh(core_axis_name="c", subcore_axis_name="s")

    @functools.partial(
        pl.kernel,
        mesh=mesh,
        out_type=jax.ShapeDtypeStruct((n_rows, d), jnp.float32),
        scratch_types=[
            pltpu.VMEM((rows_per_w,), jnp.int32),
            pltpu.VMEM((_CHUNK, _DPAD), jnp.float32),
            pltpu.VMEM((_CHUNK, _DPAD), jnp.float32),
            pltpu.SemaphoreType.DMA,
            pltpu.SemaphoreType.DMA,
        ],
        compiler_params=pltpu.CompilerParams(
            use_tc_tiling_on_sc=False, needs_layout_passes=False
        ),
    )
    def embed(idx_hbm, table_hbm, out_hbm, idx_v, rows0, rows1, sem0, sem1):
        wid = lax.axis_index("s") * _NC + lax.axis_index("c")
        base = wid * rows_per_w
        pltpu.sync_copy(idx_hbm.at[pl.ds(base, rows_per_w)], idx_v)

        def start_gather(c, rows_v, sem):
            pltpu.async_copy(
                table_hbm.at[idx_v.at[pl.ds(c * _CHUNK, _CHUNK)]], rows_v, sem
            )

        def finish_chunk(c, rows_v, sem):
            pltpu.make_async_copy(
                table_hbm.at[idx_v.at[pl.ds(c * _CHUNK, _CHUNK)]], rows_v, sem
            ).wait()

            def row_body(r, carry2):
                for j in range(d // _LANES):
                    sl = pl.ds(j * _LANES, _LANES)
                    rows_v[r, sl] = rows_v[r, sl] * scale
                return carry2

            lax.fori_loop(0, _CHUNK, row_body, 0, unroll=2)
            pltpu.sync_copy(
                rows_v.at[:, pl.ds(0, d)],
                out_hbm.at[pl.ds(base + c * _CHUNK, _CHUNK)],
            )

        start_gather(0, rows0, sem0)

        def pair_body(p, carry):
            c = 2 * p
            start_gather(c + 1, rows1, sem1)
            finish_chunk(c, rows0, sem0)

            @pl.when(p + 1 < n_pairs)
            def _():
                start_gather(c + 2, rows0, sem0)

            finish_chunk(c + 1, rows1, sem1)
            return carry

        lax.fori_loop(0, n_pairs, pair_body, 0)

    return embed


def kernel(x, table):
    b, s = x.shape
    vocab, d = table.shape
    n_rows = b * s
    tbl = jnp.pad(table, ((0, 0), (0, _DPAD - d)))
    rows = _make_embed(n_rows, d)(x.reshape(n_rows), tbl)
    return rows.reshape(b, s, d)


# speedup vs baseline: 1.0238x; 1.0238x over previous
import functools
import math

import jax
import jax.numpy as jnp
from jax import lax
from jax.experimental import pallas as pl
from jax.experimental.pallas import tpu as pltpu
from jax.experimental.pallas import tpu_sc as plsc

_NC = 2
_NS = 16
_NW = _NC * _NS
_LANES = 16
_SGRP = 4


def _make_embed(batch: int, seq: int, d: int):
    assert batch % (128 * _NW) == 0 and batch // 128 == _NW
    assert seq % 8 == 0 and d % _LANES == 0
    chunk = _SGRP * 128
    n_groups = seq // _SGRP
    assert n_groups % 2 == 0
    n_pairs = n_groups // 2
    scale = jnp.float32(math.sqrt(d))
    mesh = plsc.VectorSubcoreMesh(core_axis_name="c", subcore_axis_name="s")

    @functools.partial(
        pl.kernel,
        mesh=mesh,
        out_type=jax.ShapeDtypeStruct((seq, _NW, 128, d), jnp.float32),
        scratch_types=[
            pltpu.VMEM((chunk,), jnp.int32),
            pltpu.VMEM((chunk,), jnp.int32),
            pltpu.VMEM((chunk, d), jnp.float32),
            pltpu.VMEM((chunk, d), jnp.float32),
            pltpu.SemaphoreType.DMA,
            pltpu.SemaphoreType.DMA,
        ],
        compiler_params=pltpu.CompilerParams(
            use_tc_tiling_on_sc=False, needs_layout_passes=False
        ),
    )
    def embed(idx_hbm, table_hbm, out_hbm, idx0, idx1, rows0, rows1, sem0, sem1):
        wid = lax.axis_index("s") * _NC + lax.axis_index("c")

        def start_gather(g, idx_v, rows_v, sem):
            pltpu.sync_copy(
                idx_hbm.at[g // 2, wid, pl.ds((g % 2) * chunk, chunk)], idx_v
            )
            pltpu.async_copy(table_hbm.at[idx_v], rows_v, sem)

        def finish_group(g, idx_v, rows_v, sem):
            pltpu.make_async_copy(table_hbm.at[idx_v], rows_v, sem).wait()

            def row_body(r, carry2):
                for j in range(d // _LANES):
                    sl = pl.ds(j * _LANES, _LANES)
                    rows_v[r, sl] = rows_v[r, sl] * scale
                return carry2

            lax.fori_loop(0, chunk, row_body, 0, unroll=2)
            for q in range(_SGRP):
                pltpu.sync_copy(
                    rows_v.at[pl.ds(q * 128, 128)],
                    out_hbm.at[g * _SGRP + q, wid],
                )

        start_gather(0, idx0, rows0, sem0)

        def pair_body(p, carry):
            g = 2 * p
            start_gather(g + 1, idx1, rows1, sem1)
            finish_group(g, idx0, rows0, sem0)

            @pl.when(p + 1 < n_pairs)
            def _():
                start_gather(g + 2, idx0, rows0, sem0)

            finish_group(g + 1, idx1, rows1, sem1)
            return carry

        lax.fori_loop(0, n_pairs, pair_body, 0)

    return embed


def kernel(x, table):
    b, s = x.shape
    vocab, d = table.shape
    x4 = (
        x.T.reshape(s // 8, 8, b // 128, 128)
        .transpose(0, 2, 1, 3)
        .reshape(s // 8, b // 128, 1024)
    )
    rows = _make_embed(b, s, d)(x4, table)
    return rows.reshape(s, b, d).transpose(1, 0, 2)
